# baseline (speedup 1.0000x reference)
import functools
import math

import jax
import jax.numpy as jnp
from jax import lax
from jax.experimental import pallas as pl
from jax.experimental.pallas import tpu as pltpu
from jax.experimental.pallas import tpu_sc as plsc

N = 10000
NPAD = 10240
E = 320000
EPAD = 327680
SEQ_DIM = 9216
H = 64
BM = 2000
BK = 2304
NK = SEQ_DIM // BK
C = 128
TILES = 32
EPT = EPAD // TILES
NCH = EPT // C
RPT = NPAD // 16
BM2 = 1000

_INV_SQRT2 = 1.0 / math.sqrt(2.0)


def _gelu(v):
    return 0.5 * v * (1.0 + lax.erf(v * _INV_SQRT2))


def _encode_body(seq_ref, wenc_ref, x_ref, p_ref, benc_ref, wfc_ref, bfc_ref,
                 out_ref, acc_ref):
    k = pl.program_id(1)

    @pl.when(k == 0)
    def _():
        acc_ref[...] = jnp.zeros_like(acc_ref)

    acc_ref[...] += jnp.dot(seq_ref[...], wenc_ref[...],
                            preferred_element_type=jnp.float32)

    @pl.when(k == NK - 1)
    def _():
        enc = _gelu(acc_ref[...] + benc_ref[...])
        wfc = wfc_ref[...]
        h1 = x_ref[...] * wfc + bfc_ref[...]
        p = p_ref[...] * wfc + bfc_ref[...]
        out_ref[...] = _gelu(jnp.concatenate([p, h1 + enc], axis=1))


def _encode(seq, wenc_t, x2, pause2, benc2, wfc2, bfc2):
    return pl.pallas_call(
        _encode_body,
        grid=(N // BM, NK),
        in_specs=[
            pl.BlockSpec((BM, BK), lambda m, k: (m, k)),
            pl.BlockSpec((BK, 32), lambda m, k: (k, 0)),
            pl.BlockSpec((BM, 1), lambda m, k: (m, 0)),
            pl.BlockSpec((BM, 1), lambda m, k: (m, 0)),
            pl.BlockSpec((1, 32), lambda m, k: (0, 0)),
            pl.BlockSpec((1, 32), lambda m, k: (0, 0)),
            pl.BlockSpec((1, 32), lambda m, k: (0, 0)),
        ],
        out_specs=pl.BlockSpec((BM, H), lambda m, k: (m, 0)),
        out_shape=jax.ShapeDtypeStruct((NPAD, H), jnp.float32),
        scratch_shapes=[pltpu.VMEM((BM, 32), jnp.float32)],
    )(seq, wenc_t, x2, pause2, benc2, wfc2, bfc2)


_K = 2
NCH0 = 120
NCH1 = 40
NCHS = NCH0 + NCH1


def _sage_agg_body(h_hbm, src_hbm, dst_hbm, zeros_hbm, zerod_hbm,
                   out_hbm, deg_hbm,
                   src_v, dst_v, rows_v, rows_w, ones_v, agg_sh, deg_sh,
                   gsem, ssem, dsem):
    c = lax.axis_index("c")
    s = lax.axis_index("s")

    def fill_ones(i, carry):
        ones_v[pl.ds(i * 16, 16)] = jnp.ones((16,), jnp.float32)
        return carry

    lax.fori_loop(0, C // 16, fill_ones, 0)

    r0 = pl.multiple_of(s * RPT, 8)
    pltpu.sync_copy(zeros_hbm.at[pl.ds(r0, RPT)], agg_sh.at[pl.ds(r0, RPT)])
    pltpu.sync_copy(zerod_hbm.at[pl.ds(r0, RPT)], deg_sh.at[pl.ds(r0, RPT)])
    e0 = pl.multiple_of(s * NCHS + c * NCH0, 8)
    banks = (rows_v, rows_w)

    def run_pipeline(nch_static):
        pltpu.sync_copy(src_hbm.at[pl.ds(e0, nch_static)],
                        src_v.at[pl.ds(0, nch_static)])
        pltpu.sync_copy(dst_hbm.at[pl.ds(e0, nch_static)],
                        dst_v.at[pl.ds(0, nch_static)])

        def fire_gathers(g, bank):
            for j in range(_K):
                pltpu.async_copy(h_hbm.at[src_v.at[g * _K + j]],
                                 banks[bank].at[j], gsem)

        def process(g, bank):
            for j in range(_K):
                pltpu.make_async_copy(h_hbm.at[src_v.at[g * _K + j]],
                                      banks[bank].at[j], gsem).wait()
            for j in range(_K):
                pltpu.async_copy(banks[bank].at[j],
                                 agg_sh.at[dst_v.at[g * _K + j]], ssem,
                                 add=True)
                pltpu.async_copy(ones_v,
                                 deg_sh.at[dst_v.at[g * _K + j]], dsem,
                                 add=True)
            for j in range(_K):
                pltpu.make_async_copy(banks[bank].at[j],
                                      agg_sh.at[dst_v.at[g * _K + j]],
                                      ssem).wait()

        ngp = nch_static // _K // 2
        fire_gathers(0, 0)

        def body(gp, carry):
            g0 = gp * 2
            fire_gathers(g0 + 1, 1)
            process(g0, 0)
            fire_gathers(g0 + 2, 0)
            process(g0 + 1, 1)
            return carry

        lax.fori_loop(0, ngp - 1, body, 0)
        g0 = 2 * (ngp - 1)
        fire_gathers(g0 + 1, 1)
        process(g0, 0)
        process(g0 + 1, 1)

        def drain_deg(g, carry):
            pltpu.make_async_copy(ones_v, deg_sh.at[dst_v.at[g]], dsem).wait()
            return carry

        lax.fori_loop(0, nch_static, drain_deg, 0)

    @pl.when(c == 0)
    def _():
        run_pipeline(NCH0)

    @pl.when(c == 1)
    def _():
        run_pipeline(NCH1)

    plsc.subcore_barrier()
    pltpu.sync_copy(agg_sh.at[pl.ds(r0, RPT)], out_hbm.at[c, pl.ds(r0, RPT)])
    pltpu.sync_copy(deg_sh.at[pl.ds(r0, RPT)], deg_hbm.at[c, pl.ds(r0, RPT)])


def _sage_agg(h_pad, src2d, dst2d, zeros, zerod):
    mesh = plsc.VectorSubcoreMesh(core_axis_name="c", subcore_axis_name="s")
    f = functools.partial(
        pl.kernel,
        mesh=mesh,
        out_type=[
            jax.ShapeDtypeStruct((2, NPAD, H), jnp.float32),
            jax.ShapeDtypeStruct((2, NPAD), jnp.float32),
        ],
        scratch_types=[
            pltpu.VMEM((NCH0, C), jnp.int32),
            pltpu.VMEM((NCH0, C), jnp.int32),
            pltpu.VMEM((_K, C, H), jnp.float32),
            pltpu.VMEM((_K, C, H), jnp.float32),
            pltpu.VMEM((C,), jnp.float32),
            pltpu.VMEM_SHARED((NPAD, H), jnp.float32),
            pltpu.VMEM_SHARED((NPAD,), jnp.float32),
            pltpu.SemaphoreType.DMA,
            pltpu.SemaphoreType.DMA,
            pltpu.SemaphoreType.DMA,
        ],
        compiler_params=pltpu.CompilerParams(use_tc_tiling_on_sc=False),
    )(_sage_agg_body)
    return f(h_pad, src2d, dst2d, zeros, zerod)


def _finish_body(parts_ref, degp_ref, h_ref, wl_ref, bl_ref, wr_ref, wreg_ref,
                 breg_ref, z_ref, y_ref):
    tot = parts_ref[0] + parts_ref[1]
    deg = degp_ref[0] + degp_ref[1]
    scaled = tot / jnp.maximum(deg, 1.0)
    out = (jnp.dot(scaled, wl_ref[...], preferred_element_type=jnp.float32)
           + bl_ref[...]
           + jnp.dot(h_ref[...], wr_ref[...], preferred_element_type=jnp.float32))
    z = _gelu(out)
    z_ref[...] = z
    y_ref[...] = jnp.sum(z * wreg_ref[...], axis=1, keepdims=True) + breg_ref[...]


def _finish(parts, degp, h_pad, wl_t, bl2, wr_t, wreg2, breg2):
    return pl.pallas_call(
        _finish_body,
        grid=(N // BM2,),
        in_specs=[
            pl.BlockSpec((2, BM2, H), lambda m: (0, m, 0)),
            pl.BlockSpec((2, BM2, 1), lambda m: (0, m, 0)),
            pl.BlockSpec((BM2, H), lambda m: (m, 0)),
            pl.BlockSpec((H, 32), lambda m: (0, 0)),
            pl.BlockSpec((1, 32), lambda m: (0, 0)),
            pl.BlockSpec((H, 32), lambda m: (0, 0)),
            pl.BlockSpec((1, 32), lambda m: (0, 0)),
            pl.BlockSpec((1, 1), lambda m: (0, 0)),
        ],
        out_specs=[
            pl.BlockSpec((BM2, 32), lambda m: (m, 0)),
            pl.BlockSpec((BM2, 1), lambda m: (m, 0)),
        ],
        out_shape=[
            jax.ShapeDtypeStruct((N, 32), jnp.float32),
            jax.ShapeDtypeStruct((N, 1), jnp.float32),
        ],
    )(parts, degp, h_pad, wl_t, bl2, wr_t, wreg2, breg2)


def kernel(x, seq, pause, edge_index, W_fc, b_fc, W_enc, b_enc, W_l, b_l,
           W_r, W_reg, b_reg):
    x2 = x.reshape(N, 1)
    pause2 = pause.reshape(N, 1)
    wenc_t = W_enc.T
    wfc2 = W_fc.reshape(1, 32)
    bfc2 = b_fc.reshape(1, 32)
    benc2 = b_enc.reshape(1, 32)

    h_pad = _encode(seq, wenc_t, x2, pause2, benc2, wfc2, bfc2)

    fill = jnp.full((EPAD - E,), N, dtype=jnp.int32)
    src2d = jnp.concatenate([edge_index[0], fill]).reshape(EPAD // C, C)
    dst2d = jnp.concatenate([edge_index[1], fill]).reshape(EPAD // C, C)
    zeros = jnp.zeros((NPAD, H), jnp.float32)
    zerod = jnp.zeros((NPAD,), jnp.float32)

    parts, degs = _sage_agg(h_pad, src2d, dst2d, zeros, zerod)

    wl_t = W_l.T
    wr_t = W_r.T
    bl2 = b_l.reshape(1, 32)
    wreg2 = W_reg.reshape(1, 32)
    breg2 = b_reg.reshape(1, 1)
    degp = degs.reshape(2, NPAD, 1)

    z, y = _finish(parts, degp, h_pad, wl_t, bl2, wr_t, wreg2, breg2)
    return (y, z)


# speedup vs baseline: 1.1258x; 1.1258x over previous
import functools
import math

import jax
import jax.numpy as jnp
from jax import lax
from jax.experimental import pallas as pl
from jax.experimental.pallas import tpu as pltpu
from jax.experimental.pallas import tpu_sc as plsc

N = 10000
NPAD = 10240
E = 320000
EPAD = 327680
SEQ_DIM = 9216
H = 64
BM = 2000
BK = 2304
NK = SEQ_DIM // BK
C = 128
TILES = 32
EPT = EPAD // TILES
NCH = EPT // C
RPT = NPAD // 16
BM2 = 1000

_INV_SQRT2 = 1.0 / math.sqrt(2.0)


def _gelu(v):
    return 0.5 * v * (1.0 + lax.erf(v * _INV_SQRT2))


def _encode_body(seq_ref, wenc_ref, x_ref, p_ref, benc_ref, wfc_ref, bfc_ref,
                 out_ref, out2_ref, acc_ref):
    k = pl.program_id(1)

    @pl.when(k == 0)
    def _():
        acc_ref[...] = jnp.zeros_like(acc_ref)

    acc_ref[...] += jnp.dot(seq_ref[...], wenc_ref[...],
                            preferred_element_type=jnp.float32)

    @pl.when(k == NK - 1)
    def _():
        enc = _gelu(acc_ref[...] + benc_ref[...])
        wfc = wfc_ref[...]
        h1 = x_ref[...] * wfc + bfc_ref[...]
        p = p_ref[...] * wfc + bfc_ref[...]
        hv = _gelu(jnp.concatenate([p, h1 + enc], axis=1))
        out_ref[...] = hv
        out2_ref[...] = hv


def _encode(seq, wenc_t, x2, pause2, benc2, wfc2, bfc2):
    return pl.pallas_call(
        _encode_body,
        grid=(N // BM, NK),
        in_specs=[
            pl.BlockSpec((BM, BK), lambda m, k: (m, k)),
            pl.BlockSpec((BK, 32), lambda m, k: (k, 0)),
            pl.BlockSpec((BM, 1), lambda m, k: (m, 0)),
            pl.BlockSpec((BM, 1), lambda m, k: (m, 0)),
            pl.BlockSpec((1, 32), lambda m, k: (0, 0)),
            pl.BlockSpec((1, 32), lambda m, k: (0, 0)),
            pl.BlockSpec((1, 32), lambda m, k: (0, 0)),
        ],
        out_specs=[pl.BlockSpec((BM, H), lambda m, k: (m, 0)),
                   pl.BlockSpec((BM, H), lambda m, k: (m, 0))],
        out_shape=[jax.ShapeDtypeStruct((NPAD, H), jnp.float32),
                   jax.ShapeDtypeStruct((NPAD, H), jnp.float32)],
        scratch_shapes=[pltpu.VMEM((BM, 32), jnp.float32)],
    )(seq, wenc_t, x2, pause2, benc2, wfc2, bfc2)


_K = 2
NCH0 = 120
NCH1 = 40
NCHS = NCH0 + NCH1


def _sage_agg_body(h_hbm, h2_hbm, src_hbm, dst_hbm, zeros_hbm, zerod_hbm,
                   out_hbm, deg_hbm,
                   src_v, dst_v, rows_v, rows_w, ones_v, agg_sh, deg_sh,
                   gsem, ssem, dsem):
    c = lax.axis_index("c")
    s = lax.axis_index("s")

    def fill_ones(i, carry):
        ones_v[pl.ds(i * 16, 16)] = jnp.ones((16,), jnp.float32)
        return carry

    lax.fori_loop(0, C // 16, fill_ones, 0)

    r0 = pl.multiple_of(s * RPT, 8)
    pltpu.sync_copy(zeros_hbm.at[pl.ds(r0, RPT)], agg_sh.at[pl.ds(r0, RPT)])
    pltpu.sync_copy(zerod_hbm.at[pl.ds(r0, RPT)], deg_sh.at[pl.ds(r0, RPT)])
    e0 = pl.multiple_of(s * NCHS + c * NCH0, 8)
    banks = (rows_v, rows_w)

    def run_pipeline(nch_static, hsrc):
        pltpu.sync_copy(src_hbm.at[pl.ds(e0, nch_static)],
                        src_v.at[pl.ds(0, nch_static)])
        pltpu.sync_copy(dst_hbm.at[pl.ds(e0, nch_static)],
                        dst_v.at[pl.ds(0, nch_static)])

        def fire_gathers(g, bank):
            for j in range(_K):
                pltpu.async_copy(hsrc.at[src_v.at[g * _K + j]],
                                 banks[bank].at[j], gsem)

        def process(g, bank):
            for j in range(_K):
                pltpu.make_async_copy(hsrc.at[src_v.at[g * _K + j]],
                                      banks[bank].at[j], gsem).wait()
            for j in range(_K):
                pltpu.async_copy(banks[bank].at[j],
                                 agg_sh.at[dst_v.at[g * _K + j]], ssem,
                                 add=True)
                pltpu.async_copy(ones_v,
                                 deg_sh.at[dst_v.at[g * _K + j]], dsem,
                                 add=True)
            for j in range(_K):
                pltpu.make_async_copy(banks[bank].at[j],
                                      agg_sh.at[dst_v.at[g * _K + j]],
                                      ssem).wait()

        ngp = nch_static // _K // 2
        fire_gathers(0, 0)

        def body(gp, carry):
            g0 = gp * 2
            fire_gathers(g0 + 1, 1)
            process(g0, 0)
            fire_gathers(g0 + 2, 0)
            process(g0 + 1, 1)
            return carry

        lax.fori_loop(0, ngp - 1, body, 0)
        g0 = 2 * (ngp - 1)
        fire_gathers(g0 + 1, 1)
        process(g0, 0)
        process(g0 + 1, 1)

        def drain_deg(g, carry):
            pltpu.make_async_copy(ones_v, deg_sh.at[dst_v.at[g]], dsem).wait()
            return carry

        lax.fori_loop(0, nch_static, drain_deg, 0)

    @pl.when(c == 0)
    def _():
        run_pipeline(NCH0, h_hbm)

    @pl.when(c == 1)
    def _():
        run_pipeline(NCH1, h2_hbm)

    plsc.subcore_barrier()
    pltpu.sync_copy(agg_sh.at[pl.ds(r0, RPT)], out_hbm.at[c, pl.ds(r0, RPT)])
    pltpu.sync_copy(deg_sh.at[pl.ds(r0, RPT)], deg_hbm.at[c, pl.ds(r0, RPT)])


def _sage_agg(h_pad, h_pad2, src2d, dst2d, zeros, zerod):
    mesh = plsc.VectorSubcoreMesh(core_axis_name="c", subcore_axis_name="s")
    f = functools.partial(
        pl.kernel,
        mesh=mesh,
        out_type=[
            jax.ShapeDtypeStruct((2, NPAD, H), jnp.float32),
            jax.ShapeDtypeStruct((2, NPAD), jnp.float32),
        ],
        scratch_types=[
            pltpu.VMEM((NCH0, C), jnp.int32),
            pltpu.VMEM((NCH0, C), jnp.int32),
            pltpu.VMEM((_K, C, H), jnp.float32),
            pltpu.VMEM((_K, C, H), jnp.float32),
            pltpu.VMEM((C,), jnp.float32),
            pltpu.VMEM_SHARED((NPAD, H), jnp.float32),
            pltpu.VMEM_SHARED((NPAD,), jnp.float32),
            pltpu.SemaphoreType.DMA,
            pltpu.SemaphoreType.DMA,
            pltpu.SemaphoreType.DMA,
        ],
        compiler_params=pltpu.CompilerParams(use_tc_tiling_on_sc=False),
    )(_sage_agg_body)
    return f(h_pad, h_pad2, src2d, dst2d, zeros, zerod)


def _finish_body(parts_ref, degp_ref, h_ref, wl_ref, bl_ref, wr_ref, wreg_ref,
                 breg_ref, z_ref, y_ref):
    tot = parts_ref[0] + parts_ref[1]
    deg = degp_ref[0] + degp_ref[1]
    scaled = tot / jnp.maximum(deg, 1.0)
    out = (jnp.dot(scaled, wl_ref[...], preferred_element_type=jnp.float32)
           + bl_ref[...]
           + jnp.dot(h_ref[...], wr_ref[...], preferred_element_type=jnp.float32))
    z = _gelu(out)
    z_ref[...] = z
    y_ref[...] = jnp.sum(z * wreg_ref[...], axis=1, keepdims=True) + breg_ref[...]


def _finish(parts, degp, h_pad, wl_t, bl2, wr_t, wreg2, breg2):
    return pl.pallas_call(
        _finish_body,
        grid=(N // BM2,),
        in_specs=[
            pl.BlockSpec((2, BM2, H), lambda m: (0, m, 0)),
            pl.BlockSpec((2, BM2, 1), lambda m: (0, m, 0)),
            pl.BlockSpec((BM2, H), lambda m: (m, 0)),
            pl.BlockSpec((H, 32), lambda m: (0, 0)),
            pl.BlockSpec((1, 32), lambda m: (0, 0)),
            pl.BlockSpec((H, 32), lambda m: (0, 0)),
            pl.BlockSpec((1, 32), lambda m: (0, 0)),
            pl.BlockSpec((1, 1), lambda m: (0, 0)),
        ],
        out_specs=[
            pl.BlockSpec((BM2, 32), lambda m: (m, 0)),
            pl.BlockSpec((BM2, 1), lambda m: (m, 0)),
        ],
        out_shape=[
            jax.ShapeDtypeStruct((N, 32), jnp.float32),
            jax.ShapeDtypeStruct((N, 1), jnp.float32),
        ],
    )(parts, degp, h_pad, wl_t, bl2, wr_t, wreg2, breg2)


def kernel(x, seq, pause, edge_index, W_fc, b_fc, W_enc, b_enc, W_l, b_l,
           W_r, W_reg, b_reg):
    x2 = x.reshape(N, 1)
    pause2 = pause.reshape(N, 1)
    wenc_t = W_enc.T
    wfc2 = W_fc.reshape(1, 32)
    bfc2 = b_fc.reshape(1, 32)
    benc2 = b_enc.reshape(1, 32)

    h_pad, h_pad2 = _encode(seq, wenc_t, x2, pause2, benc2, wfc2, bfc2)

    fill = jnp.full((EPAD - E,), N, dtype=jnp.int32)
    src2d = jnp.concatenate([edge_index[0], fill]).reshape(EPAD // C, C)
    dst2d = jnp.concatenate([edge_index[1], fill]).reshape(EPAD // C, C)
    zeros = jnp.zeros((NPAD, H), jnp.float32)
    zerod = jnp.zeros((NPAD,), jnp.float32)

    parts, degs = _sage_agg(h_pad, h_pad2, src2d, dst2d, zeros, zerod)

    wl_t = W_l.T
    wr_t = W_r.T
    bl2 = b_l.reshape(1, 32)
    wreg2 = W_reg.reshape(1, 32)
    breg2 = b_reg.reshape(1, 1)
    degp = degs.reshape(2, NPAD, 1)

    z, y = _finish(parts, degp, h_pad, wl_t, bl2, wr_t, wreg2, breg2)
    return (y, z)
